# baseline (speedup 1.0000x reference)
import functools

import jax
import jax.numpy as jnp
from jax.experimental import pallas as pl
from jax.experimental.pallas import tpu as pltpu

_NUM_CLASSES = 81
_NEG_POS_RATIO = 3


def _nll_kernel(rp_ref, rt_ref, x_ref, t_ref, t4_ref, nll_out, pos_out,
                loc_out, loc_s, *, B, Tp, NP):
    b = pl.program_id(0)
    j = pl.program_id(1)

    x = x_ref[0, 0, 0]
    xT = None
    t = jnp.maximum(t_ref[0, 0, 0], 0)
    pos = t > 0
    posf = pos.astype(jnp.float32)

    nll = t.astype(jnp.float32)

    nll_out[0, 0, 0] = nll
    pos_out[0, 0, 0] = posf

    rp = rp_ref[0, 0, 0]
    rt = rt_ref[0, 0, 0]
    ad = jnp.abs(rp - rt)
    sl1 = jnp.where(ad < 1.0, 0.5 * ad * ad, ad - 0.5)
    mask4 = (t4_ref[0, 0, 0] > 0).astype(jnp.float32)
    loc_t = jnp.sum(sl1 * mask4)
    first = jnp.logical_and(b == 0, j == 0)
    loc_s[0] = jnp.where(first, 0.0, loc_s[0]) + loc_t

    last = jnp.logical_and(b == B - 1, j == NP - 1)

    @pl.when(last)
    def _():
        loc_out[0, 0] = loc_s[0]


def _mine_kernel(nll_ref, pos_ref, loc_ref, loc_out, cls_out, *, B):
    posm = pos_ref[...]
    nll_all = nll_ref[...]
    npos = jnp.sum(posm, axis=1, keepdims=True)
    K = _NEG_POS_RATIO * npos
    pnll = jnp.sum(nll_all * posm, axis=1, keepdims=True)
    v = nll_all * (1.0 - posm)
    bits = jax.lax.bitcast_convert_type(v, jnp.int32)

    def cnt_gt(th):
        return jnp.sum((bits > th).astype(jnp.float32), axis=1, keepdims=True)

    def body(_, carry):
        lo, hi = carry
        mid = lo + ((hi - lo) >> 1)
        g = cnt_gt(mid) < K
        return jnp.where(g, lo, mid + 1), jnp.where(g, mid, hi)

    lo0 = jnp.zeros((B, 1), jnp.int32)
    hi0 = jnp.full((B, 1), 0x7F800000, jnp.int32)
    _, tb = jax.lax.fori_loop(0, 31, body, (lo0, hi0))
    tv = jax.lax.bitcast_convert_type(tb, jnp.float32)
    gt = bits > tb
    cgt = jnp.sum(gt.astype(jnp.float32), axis=1, keepdims=True)
    sgt = jnp.sum(jnp.where(gt, v, 0.0), axis=1, keepdims=True)
    extra = jnp.where(K > 0.0, sgt + (K - cgt) * tv, 0.0)
    denom = jnp.sum(npos)
    cls_total = jnp.sum(pnll + extra)
    loc_out[0, 0] = loc_ref[0, 0] / denom
    cls_out[0, 0] = cls_total / denom


def kernel(reg_pred, cls_pred, reg_targets, cls_targets):
    B, P, _ = reg_pred.shape
    C = cls_pred.shape[-1]
    Tp = 10000 if P % 10000 == 0 else P
    NP = P // Tp
    tgt = cls_targets.reshape(B, NP, 1, Tp)
    tgt4 = jnp.broadcast_to(cls_targets[:, :, None], (B, P, 4)).reshape(
        B, NP, 1, 4 * Tp)
    rp4 = reg_pred.reshape(B, NP, 1, 4 * Tp)
    rt4 = reg_targets.reshape(B, NP, 1, 4 * Tp)

    nll, pos, loc = pl.pallas_call(
        functools.partial(_nll_kernel, B=B, Tp=Tp, NP=NP),
        grid=(B, NP),
        in_specs=[
            pl.BlockSpec((1, 1, 1, 4 * Tp), lambda b, j: (b, j, 0, 0)),
            pl.BlockSpec((1, 1, 1, 4 * Tp), lambda b, j: (b, j, 0, 0)),
            pl.BlockSpec((1, 1, 1, Tp * C), lambda b, j: (b, j, 0, 0)),
            pl.BlockSpec((1, 1, 1, Tp), lambda b, j: (b, j, 0, 0)),
            pl.BlockSpec((1, 1, 1, 4 * Tp), lambda b, j: (b, j, 0, 0)),
        ],
        out_specs=[
            pl.BlockSpec((1, 1, 1, Tp), lambda b, j: (b, j, 0, 0)),
            pl.BlockSpec((1, 1, 1, Tp), lambda b, j: (b, j, 0, 0)),
            pl.BlockSpec(memory_space=pltpu.SMEM),
        ],
        out_shape=[
            jax.ShapeDtypeStruct((B, NP, 1, Tp), jnp.float32),
            jax.ShapeDtypeStruct((B, NP, 1, Tp), jnp.float32),
            jax.ShapeDtypeStruct((1, 1), jnp.float32),
        ],
        scratch_shapes=[
            pltpu.SMEM((1,), jnp.float32),
        ],
    )(rp4, rt4, cls_pred.reshape(B, NP, 1, Tp * C), tgt, tgt4)

    loc_l, cls_l = pl.pallas_call(
        functools.partial(_mine_kernel, B=B),
        in_specs=[
            pl.BlockSpec((B, P), lambda: (0, 0)),
            pl.BlockSpec((B, P), lambda: (0, 0)),
            pl.BlockSpec(memory_space=pltpu.SMEM),
        ],
        out_specs=[
            pl.BlockSpec(memory_space=pltpu.SMEM),
            pl.BlockSpec(memory_space=pltpu.SMEM),
        ],
        out_shape=[
            jax.ShapeDtypeStruct((1, 1), jnp.float32),
            jax.ShapeDtypeStruct((1, 1), jnp.float32),
        ],
    )(nll.reshape(B, P), pos.reshape(B, P), loc)
    return (loc_l[0, 0], cls_l[0, 0])


# speedup vs baseline: 3.1848x; 3.1848x over previous
import functools

import jax
import jax.numpy as jnp
from jax.experimental import pallas as pl
from jax.experimental.pallas import tpu as pltpu

_NUM_CLASSES = 81
_NEG_POS_RATIO = 3


def _nll_kernel(rp_ref, rt_ref, x_ref, t_ref, m4_ref, nll_out, pos_out,
                loc_out, loc_s, *, B, P):
    b = pl.program_id(0)

    x = x_ref[0]
    xT = x.T
    t = jnp.maximum(t_ref[0, 0], 0)
    pos = t > 0
    posf = pos.astype(jnp.float32)

    m = jnp.max(xT)
    e = jnp.exp(xT - m)
    se = jnp.sum(e, axis=0)
    cls_idx = jax.lax.broadcasted_iota(jnp.int32, (_NUM_CLASSES, P), 0)
    xt = jnp.sum(jnp.where(cls_idx == t[None, :], xT, 0.0), axis=0)
    nll = jnp.log(se) + m - xt

    nll_out[0, 0] = nll.astype(jnp.bfloat16)
    pos_out[0, 0] = posf.astype(jnp.bfloat16)

    rp = rp_ref[0]
    rt = rt_ref[0]
    ad = jnp.abs(rp - rt)
    sl1 = jnp.where(ad < 1.0, 0.5 * ad * ad, ad - 0.5)
    mask4 = m4_ref[0].astype(jnp.float32)
    loc_t = jnp.sum(sl1 * mask4)
    loc_s[0] = jnp.where(b == 0, 0.0, loc_s[0]) + loc_t

    @pl.when(b == B - 1)
    def _():
        loc_out[0, 0] = loc_s[0]


def _mine_kernel(nll_ref, pos_ref, loc_ref, loc_out, cls_out, *, B):
    posm = pos_ref[...].astype(jnp.float32)
    nll_all = nll_ref[...].astype(jnp.float32)
    npos = jnp.sum(posm, axis=1, keepdims=True)
    K = _NEG_POS_RATIO * npos
    pnll = jnp.sum(nll_all * posm, axis=1, keepdims=True)
    v = nll_all * (1.0 - posm)
    bits = jax.lax.bitcast_convert_type(v, jnp.int32)

    def cnt_gt(th):
        return jnp.sum((bits > th).astype(jnp.float32), axis=1, keepdims=True)

    def body(_, carry):
        lo, hi = carry
        mid = lo + ((hi - lo) >> 1)
        g = cnt_gt(mid) < K
        return jnp.where(g, lo, mid + 1), jnp.where(g, mid, hi)

    lo0 = jnp.zeros((B, 1), jnp.int32)
    hi0 = jnp.full((B, 1), 0x7F800000, jnp.int32)
    _, tb = jax.lax.fori_loop(0, 31, body, (lo0, hi0))
    tv = jax.lax.bitcast_convert_type(tb, jnp.float32)
    gt = bits > tb
    cgt = jnp.sum(gt.astype(jnp.float32), axis=1, keepdims=True)
    sgt = jnp.sum(jnp.where(gt, v, 0.0), axis=1, keepdims=True)
    extra = jnp.where(K > 0.0, sgt + (K - cgt) * tv, 0.0)
    denom = jnp.sum(npos)
    cls_total = jnp.sum(pnll + extra)
    loc_out[0, 0] = loc_ref[0, 0] / denom
    cls_out[0, 0] = cls_total / denom


def kernel(reg_pred, cls_pred, reg_targets, cls_targets):
    B, P, _ = reg_pred.shape
    C = cls_pred.shape[-1]
    R = (4 * P) // 128
    tgt = cls_targets.reshape(B, 1, P)
    mask4 = jnp.broadcast_to(
        (cls_targets > 0).astype(jnp.int32)[:, :, None], (B, P, 4)
    ).reshape(B, R, 128)
    rp4 = reg_pred.reshape(B, R, 128)
    rt4 = reg_targets.reshape(B, R, 128)

    nll, pos, loc = pl.pallas_call(
        functools.partial(_nll_kernel, B=B, P=P),
        grid=(B,),
        in_specs=[
            pl.BlockSpec((1, R, 128), lambda b: (b, 0, 0)),
            pl.BlockSpec((1, R, 128), lambda b: (b, 0, 0)),
            pl.BlockSpec((1, P, C), lambda b: (b, 0, 0)),
            pl.BlockSpec((1, 1, P), lambda b: (b, 0, 0)),
            pl.BlockSpec((1, R, 128), lambda b: (b, 0, 0)),
        ],
        out_specs=[
            pl.BlockSpec((1, 1, P), lambda b: (b, 0, 0)),
            pl.BlockSpec((1, 1, P), lambda b: (b, 0, 0)),
            pl.BlockSpec(memory_space=pltpu.SMEM),
        ],
        out_shape=[
            jax.ShapeDtypeStruct((B, 1, P), jnp.bfloat16),
            jax.ShapeDtypeStruct((B, 1, P), jnp.bfloat16),
            jax.ShapeDtypeStruct((1, 1), jnp.float32),
        ],
        scratch_shapes=[
            pltpu.SMEM((1,), jnp.float32),
        ],
    )(rp4, rt4, cls_pred, tgt, mask4)

    loc_l, cls_l = pl.pallas_call(
        functools.partial(_mine_kernel, B=B),
        in_specs=[
            pl.BlockSpec((B, P), lambda: (0, 0)),
            pl.BlockSpec((B, P), lambda: (0, 0)),
            pl.BlockSpec(memory_space=pltpu.SMEM),
        ],
        out_specs=[
            pl.BlockSpec(memory_space=pltpu.SMEM),
            pl.BlockSpec(memory_space=pltpu.SMEM),
        ],
        out_shape=[
            jax.ShapeDtypeStruct((1, 1), jnp.float32),
            jax.ShapeDtypeStruct((1, 1), jnp.float32),
        ],
    )(nll.reshape(B, P), pos.reshape(B, P), loc)
    return (loc_l[0, 0], cls_l[0, 0])
